# baseline (speedup 1.0000x reference)
import functools

import jax
import jax.numpy as jnp
from jax import lax
from jax.experimental import pallas as pl
from jax.experimental.pallas import tpu as pltpu
from jax.experimental.pallas import tpu_sc as plsc

D_MODEL = 64
NUM_CORES = 2
NUM_SUBCORES = 16
NW = NUM_CORES * NUM_SUBCORES
CHUNK = 128
NB = 8


@functools.cache
def _make_kernel(n_chunks: int):
    mesh = plsc.VectorSubcoreMesh(core_axis_name="c", subcore_axis_name="s")

    @functools.partial(
        pl.kernel,
        mesh=mesh,
        out_type=jax.ShapeDtypeStruct((NW, n_chunks, CHUNK, D_MODEL), jnp.float32),
        scratch_types=[
            pltpu.VMEM((n_chunks, CHUNK), jnp.int32),
            pltpu.VMEM((NB, CHUNK, D_MODEL), jnp.float32),
            pltpu.SemaphoreType.DMA((NB,)),
            pltpu.SemaphoreType.DMA((NB,)),
        ],
        compiler_params=pltpu.CompilerParams(use_tc_tiling_on_sc=False),
    )
    def emb_kernel(idx_hbm, table_hbm, out_hbm, idx_v, rows, gsem, wsem):
        wid = lax.axis_index("s") * NUM_CORES + lax.axis_index("c")
        pltpu.sync_copy(idx_hbm.at[wid], idx_v)

        def gather(j, b):
            return pltpu.make_async_copy(
                table_hbm.at[idx_v.at[j]], rows.at[b], gsem.at[b])

        def write(j, b):
            return pltpu.make_async_copy(
                rows.at[b], out_hbm.at[wid, j], wsem.at[b])

        for b in range(NB):
            gather(b, b).start()

        def outer(j0, _):
            for b in range(NB):
                j = j0 * NB + b
                gather(j, b).wait()
                write(j, b).start()
                write(j, b).wait()
                gather(j + NB, b).start()
            return ()

        lax.fori_loop(0, n_chunks // NB - 1, outer, (), unroll=False)

        for b in range(NB):
            j = n_chunks - NB + b
            gather(j, b).wait()
            write(j, b).start()
        for b in range(NB):
            write(n_chunks - NB + b, b).wait()

    return emb_kernel


def kernel(inputs, table):
    batch, hist = inputs.shape
    total = batch * hist
    assert total % (NW * CHUNK) == 0
    n_chunks = total // (NW * CHUNK)
    assert n_chunks % NB == 0
    idx = inputs.astype(jnp.int32).reshape(NW, n_chunks, CHUNK)
    out = _make_kernel(n_chunks)(idx, table)
    return out.reshape(batch, hist, table.shape[1])


# speedup vs baseline: 1.2238x; 1.2238x over previous
import functools

import jax
import jax.numpy as jnp
from jax import lax
from jax.experimental import pallas as pl
from jax.experimental.pallas import tpu as pltpu
from jax.experimental.pallas import tpu_sc as plsc

D_MODEL = 64
DP = 128
NUM_CORES = 2
NUM_SUBCORES = 16
NW = NUM_CORES * NUM_SUBCORES
CHUNK = 128
NB = 5


@functools.cache
def _make_kernel(total: int):
    per_w = total // NW
    n_chunks = per_w // CHUNK
    mesh = plsc.VectorSubcoreMesh(core_axis_name="c", subcore_axis_name="s")

    @functools.partial(
        pl.kernel,
        mesh=mesh,
        out_type=jax.ShapeDtypeStruct((total, DP), jnp.float32),
        scratch_types=[
            pltpu.VMEM((per_w,), jnp.int32),
            pltpu.VMEM((NB, CHUNK, DP), jnp.float32),
            pltpu.SemaphoreType.DMA((NB,)),
            pltpu.SemaphoreType.DMA((NB,)),
        ],
    )
    def emb_kernel(idx_hbm, table_hbm, out_hbm, idx_v, rows, gsem, wsem):
        wid = lax.axis_index("s") * NUM_CORES + lax.axis_index("c")
        base = wid * per_w
        pltpu.sync_copy(idx_hbm.at[pl.ds(base, per_w)], idx_v)

        def gather(j, b):
            return pltpu.make_async_copy(
                table_hbm.at[idx_v.at[pl.ds(j * CHUNK, CHUNK)]],
                rows.at[b], gsem.at[b])

        def write(j, b):
            return pltpu.make_async_copy(
                rows.at[b],
                out_hbm.at[pl.ds(base + j * CHUNK, CHUNK)],
                wsem.at[b])

        for b in range(NB):
            gather(b, b).start()

        def outer(j0, _):
            for b in range(NB):
                j = j0 * NB + b
                gather(j, b).wait()
                write(j, b).start()
                write(j, b).wait()
                gather(j + NB, b).start()
            return ()

        lax.fori_loop(0, n_chunks // NB - 1, outer, (), unroll=False)

        for b in range(NB):
            j = n_chunks - NB + b
            gather(j, b).wait()
            write(j, b).start()
        for b in range(NB):
            write(n_chunks - NB + b, b).wait()

    return emb_kernel


def kernel(inputs, table):
    batch, hist = inputs.shape
    total = batch * hist
    assert total % (NW * CHUNK) == 0
    idx = inputs.astype(jnp.int32).reshape(total)
    table_p = jnp.pad(table, ((0, 0), (0, DP - table.shape[1])))
    out = _make_kernel(total)(idx, table_p)
    return out[:, :D_MODEL].reshape(batch, hist, table.shape[1])
